# baseline (speedup 1.0000x reference)
import functools

import jax
import jax.numpy as jnp
from jax import lax
from jax.experimental import pallas as pl
from jax.experimental.pallas import tpu as pltpu
from jax.experimental.pallas import tpu_sc as plsc

_NUM = 1000000
_RANK = 16
_OUT_DIM = 64
_B = 16384
_L = 50
_N = _B * _L
_PACK = 128 // _RANK
_NP = _N // _PACK

_NC = 2
_NS = 16
_NW = _NC * _NS
_PROWS_PER_W = _NP // _NW
_PCHUNK = 400
_NCHUNKS = _PROWS_PER_W // _PCHUNK


_TR_TILES = 25
_TR_COLS_PER_W = _NUM // _TR_TILES
_TR_CHUNK = 2000
_TR_NCHUNKS = _TR_COLS_PER_W // _TR_CHUNK


def _sc_transpose(table_t):
    mesh = plsc.VectorSubcoreMesh(
        core_axis_name="c", subcore_axis_name="s", num_cores=_NC,
        num_subcores=_NS)

    @functools.partial(
        pl.kernel,
        out_type=jax.ShapeDtypeStruct((_NUM * _RANK // 128, 128),
                                      jnp.float32),
        mesh=mesh,
        scratch_types=[
            pltpu.VMEM((_RANK, _TR_CHUNK), jnp.float32),
            pltpu.VMEM((_TR_CHUNK // _PACK, 128), jnp.float32),
        ],
        compiler_params=pltpu.CompilerParams(
            use_tc_tiling_on_sc=False, needs_layout_passes=False),
    )
    def transpose_kernel(tt_hbm, out_hbm, strips_v, outv):
        wid = lax.axis_index("s") * _NC + lax.axis_index("c")

        @pl.when(wid < _TR_TILES)
        def _():
            lanes = lax.iota(jnp.int32, 16)
            for c in range(_TR_NCHUNKS):
                col0 = wid * _TR_COLS_PER_W + c * _TR_CHUNK
                for k in range(_RANK):
                    pltpu.sync_copy(
                        tt_hbm.at[k, pl.ds(col0, _TR_CHUNK)],
                        strips_v.at[k])

                def body(p, _):
                    for j in range(_PACK):
                        v = plsc.load_gather(
                            strips_v,
                            [lanes, jnp.full((16,), p * _PACK + j,
                                             jnp.int32)])
                        outv[p, pl.ds(j * _RANK, _RANK)] = v
                    return _

                lax.fori_loop(0, _TR_CHUNK // _PACK, body, 0)
                prow0 = pl.multiple_of(col0 // _PACK,
                                       _TR_CHUNK // _PACK)
                pltpu.sync_copy(
                    outv, out_hbm.at[pl.ds(prow0, _TR_CHUNK // _PACK)])

    return transpose_kernel(table_t)


def _sc_gather(emb_table, idx):
    mesh = plsc.VectorSubcoreMesh(
        core_axis_name="c", subcore_axis_name="s", num_cores=_NC,
        num_subcores=_NS)

    @functools.partial(
        pl.kernel,
        out_type=jax.ShapeDtypeStruct((_NP, 128), jnp.float32),
        mesh=mesh,
        scratch_types=[
            pltpu.VMEM((_PACK, _PCHUNK), jnp.int32),
            pltpu.VMEM((_PACK, _PCHUNK, _RANK), jnp.float32),
            pltpu.SemaphoreType.DMA,
            pltpu.SemaphoreType.DMA,
        ],
        compiler_params=pltpu.CompilerParams(use_tc_tiling_on_sc=False),
    )
    def gather_kernel(table_hbm, idx_hbm, out_hbm, idx_v, rows_v, sem,
                      out_sem):
        wid = lax.axis_index("s") * _NC + lax.axis_index("c")
        base = wid * _PROWS_PER_W
        for c in range(_NCHUNKS):
            p0 = base + c * _PCHUNK
            pltpu.sync_copy(idx_hbm.at[:, pl.ds(p0, _PCHUNK)], idx_v)
            copies = [
                pltpu.async_copy(
                    table_hbm.at[idx_v.at[j]], rows_v.at[j], sem)
                for j in range(_PACK)
            ]
            for cp in copies:
                cp.wait()
            writes = [
                pltpu.async_copy(
                    rows_v.at[j],
                    out_hbm.at[pl.ds(p0, _PCHUNK),
                               pl.ds(j * _RANK, _RANK)],
                    out_sem)
                for j in range(_PACK)
            ]
            for wr in writes:
                wr.wait()

    return gather_kernel(emb_table, idx)


_TC_ROWS = 1600
_BPS = _TC_ROWS // _L


def _tc_matmul_kernel(emb_ref, w_ref, out_ref):
    emb = emb_ref[...]
    for j in range(_PACK):
        r = lax.dot_general(
            emb, w_ref[j],
            dimension_numbers=(((1,), (0,)), ((), ())),
            preferred_element_type=jnp.float32)
        out_ref[j] = r.reshape(_BPS, _L, _OUT_DIM)


def _tc_project(packed, W):
    eye = jnp.eye(_PACK, dtype=W.dtype)
    w8 = jnp.einsum('jk,ro->jkro', eye, W.T).reshape(_PACK, 128, _OUT_DIM)
    grid_i = _NP // _TC_ROWS
    out4 = pl.pallas_call(
        _tc_matmul_kernel,
        grid=(grid_i,),
        in_specs=[
            pl.BlockSpec((_TC_ROWS, 128), lambda i: (i, 0)),
            pl.BlockSpec((_PACK, 128, _OUT_DIM), lambda i: (0, 0, 0)),
        ],
        out_specs=pl.BlockSpec(
            (_PACK, _BPS, _L, _OUT_DIM), lambda i: (0, i, 0, 0)),
        out_shape=jax.ShapeDtypeStruct(
            (_PACK, _B // _PACK, _L, _OUT_DIM), jnp.float32),
    )(packed, w8)
    return out4.reshape(_B, _L, _OUT_DIM)


def kernel(src, emb_table, W):
    idx2 = src.reshape(_PACK, _NP)
    tbl = _sc_transpose(jnp.swapaxes(emb_table, 0, 1))
    tbl = tbl.reshape(_NUM, _RANK)
    packed = _sc_gather(tbl, idx2)
    return _tc_project(packed, W)


# speedup vs baseline: 2.5107x; 2.5107x over previous
import functools

import jax
import jax.numpy as jnp
from jax import lax
from jax.experimental import pallas as pl
from jax.experimental.pallas import tpu as pltpu
from jax.experimental.pallas import tpu_sc as plsc

_NUM = 1000000
_RANK = 16
_OUT_DIM = 64
_B = 16384
_L = 50
_N = _B * _L
_PACK = 128 // _RANK
_NP = _N // _PACK

_NC = 2
_NS = 16
_NW = _NC * _NS
_PROWS_PER_W = _NP // _NW
_PCHUNK = 400
_NCHUNKS = _PROWS_PER_W // _PCHUNK


def _sc_gather(emb_table, idx):
    mesh = plsc.VectorSubcoreMesh(
        core_axis_name="c", subcore_axis_name="s", num_cores=_NC,
        num_subcores=_NS)

    @functools.partial(
        pl.kernel,
        out_type=jax.ShapeDtypeStruct((_NP, 128), jnp.float32),
        mesh=mesh,
        scratch_types=[
            pltpu.VMEM((_PACK, _PCHUNK), jnp.int32),
            pltpu.VMEM((_PACK, _PCHUNK, _RANK), jnp.float32),
            pltpu.SemaphoreType.DMA,
            pltpu.SemaphoreType.DMA,
        ],
        compiler_params=pltpu.CompilerParams(use_tc_tiling_on_sc=False),
    )
    def gather_kernel(table_hbm, idx_hbm, out_hbm, idx_v, rows_v, sem,
                      out_sem):
        wid = lax.axis_index("s") * _NC + lax.axis_index("c")
        base = wid * _PROWS_PER_W
        for c in range(_NCHUNKS):
            p0 = base + c * _PCHUNK
            pltpu.sync_copy(idx_hbm.at[:, pl.ds(p0, _PCHUNK)], idx_v)
            copies = [
                pltpu.async_copy(
                    table_hbm.at[idx_v.at[j]], rows_v.at[j], sem)
                for j in range(_PACK)
            ]
            for cp in copies:
                cp.wait()
            writes = [
                pltpu.async_copy(
                    rows_v.at[j],
                    out_hbm.at[pl.ds(p0, _PCHUNK),
                               pl.ds(j * _RANK, _RANK)],
                    out_sem)
                for j in range(_PACK)
            ]
            for wr in writes:
                wr.wait()

    return gather_kernel(emb_table, idx)


_TC_ROWS = 1600
_BPS = _TC_ROWS // _L


def _tc_matmul_kernel(emb_ref, w_ref, out_ref):
    emb = emb_ref[...]
    for j in range(_PACK):
        r = lax.dot_general(
            emb, w_ref[j],
            dimension_numbers=(((1,), (0,)), ((), ())),
            preferred_element_type=jnp.float32)
        out_ref[j] = r.reshape(_BPS, _L, _OUT_DIM)


def _tc_project(packed, W):
    eye = jnp.eye(_PACK, dtype=W.dtype)
    w8 = jnp.einsum('jk,ro->jkro', eye, W.T).reshape(_PACK, 128, _OUT_DIM)
    grid_i = _NP // _TC_ROWS
    out4 = pl.pallas_call(
        _tc_matmul_kernel,
        grid=(grid_i,),
        in_specs=[
            pl.BlockSpec((_TC_ROWS, 128), lambda i: (i, 0)),
            pl.BlockSpec((_PACK, 128, _OUT_DIM), lambda i: (0, 0, 0)),
        ],
        out_specs=pl.BlockSpec(
            (_PACK, _BPS, _L, _OUT_DIM), lambda i: (0, i, 0, 0)),
        out_shape=jax.ShapeDtypeStruct(
            (_PACK, _B // _PACK, _L, _OUT_DIM), jnp.float32),
    )(packed, w8)
    return out4.reshape(_B, _L, _OUT_DIM)


def kernel(src, emb_table, W):
    idx2 = src.reshape(_PACK, _NP)
    packed = _sc_gather(emb_table, idx2)
    return _tc_project(packed, W)


# speedup vs baseline: 2.5326x; 1.0087x over previous
import functools

import jax
import jax.numpy as jnp
from jax import lax
from jax.experimental import pallas as pl
from jax.experimental.pallas import tpu as pltpu
from jax.experimental.pallas import tpu_sc as plsc

_NUM = 1000000
_RANK = 16
_OUT_DIM = 64
_B = 16384
_L = 50
_N = _B * _L
_PACK = 128 // _RANK
_NP = _N // _PACK

_NC = 2
_NS = 16
_NW = _NC * _NS
_PROWS_PER_W = _NP // _NW
_PCHUNK = 800
_NCHUNKS = _PROWS_PER_W // _PCHUNK


def _sc_gather(emb_table, idx):
    mesh = plsc.VectorSubcoreMesh(
        core_axis_name="c", subcore_axis_name="s", num_cores=_NC,
        num_subcores=_NS)

    @functools.partial(
        pl.kernel,
        out_type=jax.ShapeDtypeStruct((_NP, 128), jnp.float32),
        mesh=mesh,
        scratch_types=[
            pltpu.VMEM((_PACK, _PCHUNK), jnp.int32),
            pltpu.VMEM((_PACK, _PCHUNK, _RANK), jnp.float32),
            pltpu.SemaphoreType.DMA,
            pltpu.SemaphoreType.DMA,
        ],
        compiler_params=pltpu.CompilerParams(use_tc_tiling_on_sc=False),
    )
    def gather_kernel(table_hbm, idx_hbm, out_hbm, idx_v, rows_v, sem,
                      out_sem):
        wid = lax.axis_index("s") * _NC + lax.axis_index("c")
        base = wid * _PROWS_PER_W
        for c in range(_NCHUNKS):
            p0 = base + c * _PCHUNK
            pltpu.sync_copy(idx_hbm.at[:, pl.ds(p0, _PCHUNK)], idx_v)
            copies = [
                pltpu.async_copy(
                    table_hbm.at[idx_v.at[j]], rows_v.at[j], sem)
                for j in range(_PACK)
            ]
            for cp in copies:
                cp.wait()
            writes = [
                pltpu.async_copy(
                    rows_v.at[j],
                    out_hbm.at[pl.ds(p0, _PCHUNK),
                               pl.ds(j * _RANK, _RANK)],
                    out_sem)
                for j in range(_PACK)
            ]
            for wr in writes:
                wr.wait()

    return gather_kernel(emb_table, idx)


_TC_ROWS = 1600
_BPS = _TC_ROWS // _L


def _tc_matmul_kernel(emb_ref, w_ref, out_ref):
    emb = emb_ref[...]
    for j in range(_PACK):
        r = lax.dot_general(
            emb, w_ref[j],
            dimension_numbers=(((1,), (0,)), ((), ())),
            preferred_element_type=jnp.float32)
        out_ref[j] = r.reshape(_BPS, _L, _OUT_DIM)


def _tc_project(packed, W):
    eye = jnp.eye(_PACK, dtype=W.dtype)
    w8 = jnp.einsum('jk,ro->jkro', eye, W.T).reshape(_PACK, 128, _OUT_DIM)
    grid_i = _NP // _TC_ROWS
    out4 = pl.pallas_call(
        _tc_matmul_kernel,
        grid=(grid_i,),
        in_specs=[
            pl.BlockSpec((_TC_ROWS, 128), lambda i: (i, 0)),
            pl.BlockSpec((_PACK, 128, _OUT_DIM), lambda i: (0, 0, 0)),
        ],
        out_specs=pl.BlockSpec(
            (_PACK, _BPS, _L, _OUT_DIM), lambda i: (0, i, 0, 0)),
        out_shape=jax.ShapeDtypeStruct(
            (_PACK, _B // _PACK, _L, _OUT_DIM), jnp.float32),
    )(packed, w8)
    return out4.reshape(_B, _L, _OUT_DIM)


def kernel(src, emb_table, W):
    idx2 = src.reshape(_PACK, _NP)
    packed = _sc_gather(emb_table, idx2)
    return _tc_project(packed, W)
